# baseline (speedup 1.0000x reference)
import functools

import jax
import jax.numpy as jnp
from jax import lax
from jax.experimental import pallas as pl
from jax.experimental.pallas import tpu as pltpu

_F32 = jnp.float32
_BF16 = jnp.bfloat16

_MC = 256
_POOL_CHAINS = 2
_HEAD_TILE = 4096


def _const_spec(a):
    return pl.BlockSpec(a.shape, lambda b: (0,) * a.ndim)


def _stn_kernel(x_ref, w1_ref, b1_ref, w2_ref, b2_ref, w3_ref, b3_ref,
                fw1_ref, fb1_ref, fw2_ref, fb2_ref, fw3_ref, fb3_ref,
                o_ref, *, n, bs):
    nc = 1024 // _MC
    tnh = n // _POOL_CHAINS
    w1, b1 = w1_ref[...], b1_ref[...]
    w2, b2 = w2_ref[...], b2_ref[...]
    w3 = w3_ref[...]

    def chain(j, lo):
        xt = x_ref[j, :, pl.ds(lo, tnh)]
        h1 = jnp.maximum(
            jnp.dot(w1, xt, preferred_element_type=_F32) + b1, 0.0)
        h2 = jnp.maximum(
            jnp.dot(w2, h1, preferred_element_type=_F32) + b2, 0.0)
        return [jnp.max(
            jnp.dot(w3[c * _MC:(c + 1) * _MC, :], h2,
                    preferred_element_type=_F32),
            axis=1, keepdims=True) for c in range(nc)]

    cols = []
    for j in range(bs):
        ms = [chain(j, q * tnh) for q in range(_POOL_CHAINS)]
        cols.append(jnp.concatenate(
            [functools.reduce(jnp.maximum, parts)
             for parts in zip(*ms)], axis=0))
    g = jnp.concatenate(cols, axis=1)
    g = jnp.maximum(g + b3_ref[...], 0.0)

    h = jnp.maximum(
        jnp.dot(fw1_ref[...], g, preferred_element_type=_F32)
        + fb1_ref[...], 0.0)
    h = jnp.maximum(
        jnp.dot(fw2_ref[...], h, preferred_element_type=_F32)
        + fb2_ref[...], 0.0)
    r = (jnp.dot(fw3_ref[...], h, preferred_element_type=_F32)
         + fb3_ref[...])
    for j in range(bs):
        o_ref[j] = r[:, j:j + 1]


def _feat_head_kernel(x_ref, w1b_ref, b1_ref, w2_ref, b2_ref, w3_ref, b3_ref,
                      wg_ref, bh1_ref, wl_ref, wh2_ref, bh2_ref,
                      wh3_ref, bh3_ref, wh4_ref, bh4_ref,
                      o_ref, pf_ref, *, n, ht, nht, bs):
    nc = 1024 // _MC
    tnh = n // _POOL_CHAINS
    b1 = b1_ref[...]
    w2, b2 = w2_ref[...], b2_ref[...]
    w3 = w3_ref[...]

    def feat_chain(j, w1b, lo):
        xt = x_ref[j, :, pl.ds(lo, tnh)]
        h1 = jnp.maximum(
            jnp.dot(w1b, xt, preferred_element_type=_F32) + b1, 0.0)
        pf_ref[j, :, pl.ds(lo, tnh)] = h1
        h2 = jnp.maximum(
            jnp.dot(w2, h1, preferred_element_type=_F32) + b2, 0.0)
        return [jnp.max(
            jnp.dot(w3[c * _MC:(c + 1) * _MC, :], h2,
                    preferred_element_type=_F32),
            axis=1, keepdims=True) for c in range(nc)]

    cols = []
    for j in range(bs):
        w1b = w1b_ref[j]
        ms = [feat_chain(j, w1b, q * tnh) for q in range(_POOL_CHAINS)]
        cols.append(jnp.concatenate(
            [functools.reduce(jnp.maximum, parts)
             for parts in zip(*ms)], axis=0))
    g2 = jnp.concatenate(cols, axis=1) + b3_ref[...]

    gb = (jnp.dot(wg_ref[...], g2, preferred_element_type=_F32)
          + bh1_ref[...])

    wl = wl_ref[...]
    wh2, bh2 = wh2_ref[...], bh2_ref[...]
    wh3, bh3 = wh3_ref[...], bh3_ref[...]
    wh4, bh4 = wh4_ref[...], bh4_ref[...]

    def head_chain(j, gbj, lo):
        pf = pf_ref[j, :, pl.ds(lo, ht)]
        h = jnp.maximum(
            jnp.dot(wl, pf, preferred_element_type=_F32) + gbj, 0.0)
        h = jnp.maximum(
            jnp.dot(wh2, h, preferred_element_type=_F32) + bh2, 0.0)
        h = jnp.maximum(
            jnp.dot(wh3, h, preferred_element_type=_F32) + bh3, 0.0)
        o_ref[j, :, pl.ds(lo, ht)] = jnp.maximum(
            jnp.dot(wh4, h, preferred_element_type=_F32) + bh4, 0.0)

    for j in range(bs):
        gbj = gb[:, j:j + 1]
        for i in range(nht):
            head_chain(j, gbj, i * ht)


def kernel(x, stn_conv1_w, stn_conv1_b, stn_conv2_w, stn_conv2_b,
           stn_conv3_w, stn_conv3_b, stn_fc1_w, stn_fc1_b, stn_fc2_w,
           stn_fc2_b, stn_fc3_w, stn_fc3_b, feat_conv1_w, feat_conv1_b,
           feat_conv2_w, feat_conv2_b, feat_conv3_w, feat_conv3_b,
           head_conv1_wg, head_conv1_wl, head_conv1_b, head_conv2_w,
           head_conv2_b, head_conv3_w, head_conv3_b, head_conv4_w,
           head_conv4_b):
    B, C, N = x.shape
    k = head_conv4_w.shape[0]
    ht = _HEAD_TILE if N % _HEAD_TILE == 0 else N
    nht = N // ht
    bs = 1

    whl = head_conv1_wl

    fw1, fb1 = stn_fc1_w.T, stn_fc1_b.reshape(-1, 1)
    fw2, fb2 = stn_fc2_w.T, stn_fc2_b.reshape(-1, 1)
    fw3, fb3 = stn_fc3_w.T, stn_fc3_b.reshape(-1, 1)

    stn_in = (stn_conv1_w, stn_conv1_b, stn_conv2_w, stn_conv2_b,
              stn_conv3_w, stn_conv3_b, fw1, fb1, fw2, fb2, fw3, fb3)
    raw = pl.pallas_call(
        functools.partial(_stn_kernel, n=N, bs=bs),
        out_shape=jax.ShapeDtypeStruct((B, 36, 1), _F32),
        grid=(B // bs,),
        in_specs=[pl.BlockSpec((bs, C, N), lambda b: (b, 0, 0))]
        + [_const_spec(a) for a in stn_in],
        out_specs=pl.BlockSpec((bs, 36, 1), lambda b: (b, 0, 0)),
        compiler_params=pltpu.CompilerParams(
            dimension_semantics=("parallel",)),
    )(x, *stn_in)

    trans = raw[:, :, 0].reshape(B, C, C) + jnp.eye(C, dtype=_F32)[None]
    w1b = jnp.einsum("oc,bjc->boj", feat_conv1_w, trans)

    fh_in = (feat_conv1_b, feat_conv2_w, feat_conv2_b, feat_conv3_w,
             feat_conv3_b, head_conv1_wg, head_conv1_b, whl, head_conv2_w,
             head_conv2_b, head_conv3_w, head_conv3_b, head_conv4_w,
             head_conv4_b)
    out = pl.pallas_call(
        functools.partial(_feat_head_kernel, n=N, ht=ht, nht=nht, bs=bs),
        out_shape=jax.ShapeDtypeStruct((B, k, N), _F32),
        grid=(B // bs,),
        in_specs=[pl.BlockSpec((bs, C, N), lambda b: (b, 0, 0)),
                  pl.BlockSpec((bs, 64, C), lambda b: (b, 0, 0))]
        + [_const_spec(a) for a in fh_in],
        out_specs=pl.BlockSpec((bs, k, N), lambda b: (b, 0, 0)),
        scratch_shapes=[pltpu.VMEM((bs, 64, N), _F32)],
        compiler_params=pltpu.CompilerParams(
            dimension_semantics=("parallel",)),
    )(x, w1b, *fh_in)
    return out


# speedup vs baseline: 1.0005x; 1.0005x over previous
import functools

import jax
import jax.numpy as jnp
from jax import lax
from jax.experimental import pallas as pl
from jax.experimental.pallas import tpu as pltpu

_F32 = jnp.float32
_BF16 = jnp.bfloat16

_MC = 256
_POOL_CHAINS = 4
_HEAD_TILE = 4096


def _const_spec(a):
    return pl.BlockSpec(a.shape, lambda b: (0,) * a.ndim)


def _stn_kernel(x_ref, w1_ref, b1_ref, w2_ref, b2_ref, w3_ref, b3_ref,
                fw1_ref, fb1_ref, fw2_ref, fb2_ref, fw3_ref, fb3_ref,
                o_ref, *, n, bs):
    nc = 1024 // _MC
    tnh = n // _POOL_CHAINS
    w1, b1 = w1_ref[...], b1_ref[...]
    w2, b2 = w2_ref[...], b2_ref[...]
    w3 = w3_ref[...]

    def chain(j, lo):
        xt = x_ref[j, :, pl.ds(lo, tnh)]
        h1 = jnp.maximum(
            jnp.dot(w1, xt, preferred_element_type=_F32) + b1, 0.0)
        h2 = jnp.maximum(
            jnp.dot(w2, h1, preferred_element_type=_F32) + b2, 0.0)
        return [jnp.max(
            jnp.dot(w3[c * _MC:(c + 1) * _MC, :], h2,
                    preferred_element_type=_F32),
            axis=1, keepdims=True) for c in range(nc)]

    cols = []
    for j in range(bs):
        ms = [chain(j, q * tnh) for q in range(_POOL_CHAINS)]
        cols.append(jnp.concatenate(
            [functools.reduce(jnp.maximum, parts)
             for parts in zip(*ms)], axis=0))
    g = jnp.concatenate(cols, axis=1)
    g = jnp.maximum(g + b3_ref[...], 0.0)

    h = jnp.maximum(
        jnp.dot(fw1_ref[...], g, preferred_element_type=_F32)
        + fb1_ref[...], 0.0)
    h = jnp.maximum(
        jnp.dot(fw2_ref[...], h, preferred_element_type=_F32)
        + fb2_ref[...], 0.0)
    r = (jnp.dot(fw3_ref[...], h, preferred_element_type=_F32)
         + fb3_ref[...])
    for j in range(bs):
        o_ref[j] = r[:, j:j + 1]


def _feat_head_kernel(x_ref, w1b_ref, b1_ref, w2_ref, b2_ref, w3_ref, b3_ref,
                      wg_ref, bh1_ref, wl_ref, wh2_ref, bh2_ref,
                      wh3_ref, bh3_ref, wh4_ref, bh4_ref,
                      o_ref, pf_ref, *, n, ht, nht, bs):
    nc = 1024 // _MC
    tnh = n // _POOL_CHAINS
    b1 = b1_ref[...]
    w2, b2 = w2_ref[...], b2_ref[...]
    w3 = w3_ref[...]

    def feat_chain(j, w1b, lo):
        xt = x_ref[j, :, pl.ds(lo, tnh)]
        h1 = jnp.maximum(
            jnp.dot(w1b, xt, preferred_element_type=_F32) + b1, 0.0)
        pf_ref[j, :, pl.ds(lo, tnh)] = h1
        h2 = jnp.maximum(
            jnp.dot(w2, h1, preferred_element_type=_F32) + b2, 0.0)
        return [jnp.max(
            jnp.dot(w3[c * _MC:(c + 1) * _MC, :], h2,
                    preferred_element_type=_F32),
            axis=1, keepdims=True) for c in range(nc)]

    cols = []
    for j in range(bs):
        w1b = w1b_ref[j]
        ms = [feat_chain(j, w1b, q * tnh) for q in range(_POOL_CHAINS)]
        cols.append(jnp.concatenate(
            [functools.reduce(jnp.maximum, parts)
             for parts in zip(*ms)], axis=0))
    g2 = jnp.concatenate(cols, axis=1) + b3_ref[...]

    gb = (jnp.dot(wg_ref[...], g2, preferred_element_type=_F32)
          + bh1_ref[...])

    wl = wl_ref[...]
    wh2, bh2 = wh2_ref[...], bh2_ref[...]
    wh3, bh3 = wh3_ref[...], bh3_ref[...]
    wh4, bh4 = wh4_ref[...], bh4_ref[...]

    def head_chain(j, gbj, lo):
        pf = pf_ref[j, :, pl.ds(lo, ht)]
        h = jnp.maximum(
            jnp.dot(wl, pf, preferred_element_type=_F32) + gbj, 0.0)
        h = jnp.maximum(
            jnp.dot(wh2, h, preferred_element_type=_F32) + bh2, 0.0)
        h = jnp.maximum(
            jnp.dot(wh3, h, preferred_element_type=_F32) + bh3, 0.0)
        o_ref[j, :, pl.ds(lo, ht)] = jnp.maximum(
            jnp.dot(wh4, h, preferred_element_type=_F32) + bh4, 0.0)

    for j in range(bs):
        gbj = gb[:, j:j + 1]
        for i in range(nht):
            head_chain(j, gbj, i * ht)


def kernel(x, stn_conv1_w, stn_conv1_b, stn_conv2_w, stn_conv2_b,
           stn_conv3_w, stn_conv3_b, stn_fc1_w, stn_fc1_b, stn_fc2_w,
           stn_fc2_b, stn_fc3_w, stn_fc3_b, feat_conv1_w, feat_conv1_b,
           feat_conv2_w, feat_conv2_b, feat_conv3_w, feat_conv3_b,
           head_conv1_wg, head_conv1_wl, head_conv1_b, head_conv2_w,
           head_conv2_b, head_conv3_w, head_conv3_b, head_conv4_w,
           head_conv4_b):
    B, C, N = x.shape
    k = head_conv4_w.shape[0]
    ht = _HEAD_TILE if N % _HEAD_TILE == 0 else N
    nht = N // ht
    bs = 1

    whl = head_conv1_wl

    fw1, fb1 = stn_fc1_w.T, stn_fc1_b.reshape(-1, 1)
    fw2, fb2 = stn_fc2_w.T, stn_fc2_b.reshape(-1, 1)
    fw3, fb3 = stn_fc3_w.T, stn_fc3_b.reshape(-1, 1)

    stn_in = (stn_conv1_w, stn_conv1_b, stn_conv2_w, stn_conv2_b,
              stn_conv3_w, stn_conv3_b, fw1, fb1, fw2, fb2, fw3, fb3)
    raw = pl.pallas_call(
        functools.partial(_stn_kernel, n=N, bs=bs),
        out_shape=jax.ShapeDtypeStruct((B, 36, 1), _F32),
        grid=(B // bs,),
        in_specs=[pl.BlockSpec((bs, C, N), lambda b: (b, 0, 0))]
        + [_const_spec(a) for a in stn_in],
        out_specs=pl.BlockSpec((bs, 36, 1), lambda b: (b, 0, 0)),
        compiler_params=pltpu.CompilerParams(
            dimension_semantics=("parallel",)),
    )(x, *stn_in)

    trans = raw[:, :, 0].reshape(B, C, C) + jnp.eye(C, dtype=_F32)[None]
    w1b = jnp.einsum("oc,bjc->boj", feat_conv1_w, trans)

    fh_in = (feat_conv1_b, feat_conv2_w, feat_conv2_b, feat_conv3_w,
             feat_conv3_b, head_conv1_wg, head_conv1_b, whl, head_conv2_w,
             head_conv2_b, head_conv3_w, head_conv3_b, head_conv4_w,
             head_conv4_b)
    out = pl.pallas_call(
        functools.partial(_feat_head_kernel, n=N, ht=ht, nht=nht, bs=bs),
        out_shape=jax.ShapeDtypeStruct((B, k, N), _F32),
        grid=(B // bs,),
        in_specs=[pl.BlockSpec((bs, C, N), lambda b: (b, 0, 0)),
                  pl.BlockSpec((bs, 64, C), lambda b: (b, 0, 0))]
        + [_const_spec(a) for a in fh_in],
        out_specs=pl.BlockSpec((bs, k, N), lambda b: (b, 0, 0)),
        scratch_shapes=[pltpu.VMEM((bs, 64, N), _F32)],
        compiler_params=pltpu.CompilerParams(
            dimension_semantics=("parallel",)),
    )(x, w1b, *fh_in)
    return out


# speedup vs baseline: 1.0044x; 1.0039x over previous
import functools

import jax
import jax.numpy as jnp
from jax import lax
from jax.experimental import pallas as pl
from jax.experimental.pallas import tpu as pltpu

_F32 = jnp.float32
_BF16 = jnp.bfloat16

_MC = 256
_POOL_CHAINS = 2
_HEAD_TILE = 8192


def _const_spec(a):
    return pl.BlockSpec(a.shape, lambda b: (0,) * a.ndim)


def _stn_kernel(x_ref, w1_ref, b1_ref, w2_ref, b2_ref, w3_ref, b3_ref,
                fw1_ref, fb1_ref, fw2_ref, fb2_ref, fw3_ref, fb3_ref,
                o_ref, *, n, bs):
    nc = 1024 // _MC
    tnh = n // _POOL_CHAINS
    w1, b1 = w1_ref[...], b1_ref[...]
    w2, b2 = w2_ref[...], b2_ref[...]
    w3 = w3_ref[...]

    def chain(j, lo):
        xt = x_ref[j, :, pl.ds(lo, tnh)]
        h1 = jnp.maximum(
            jnp.dot(w1, xt, preferred_element_type=_F32) + b1, 0.0)
        h2 = jnp.maximum(
            jnp.dot(w2, h1, preferred_element_type=_F32) + b2, 0.0)
        return [jnp.max(
            jnp.dot(w3[c * _MC:(c + 1) * _MC, :], h2,
                    preferred_element_type=_F32),
            axis=1, keepdims=True) for c in range(nc)]

    cols = []
    for j in range(bs):
        ms = [chain(j, q * tnh) for q in range(_POOL_CHAINS)]
        cols.append(jnp.concatenate(
            [functools.reduce(jnp.maximum, parts)
             for parts in zip(*ms)], axis=0))
    g = jnp.concatenate(cols, axis=1)
    g = jnp.maximum(g + b3_ref[...], 0.0)

    h = jnp.maximum(
        jnp.dot(fw1_ref[...], g, preferred_element_type=_F32)
        + fb1_ref[...], 0.0)
    h = jnp.maximum(
        jnp.dot(fw2_ref[...], h, preferred_element_type=_F32)
        + fb2_ref[...], 0.0)
    r = (jnp.dot(fw3_ref[...], h, preferred_element_type=_F32)
         + fb3_ref[...])
    for j in range(bs):
        o_ref[j] = r[:, j:j + 1]


def _feat_head_kernel(x_ref, w1b_ref, b1_ref, w2_ref, b2_ref, w3_ref, b3_ref,
                      wg_ref, bh1_ref, wl_ref, wh2_ref, bh2_ref,
                      wh3_ref, bh3_ref, wh4_ref, bh4_ref,
                      o_ref, pf_ref, *, n, ht, nht, bs):
    nc = 1024 // _MC
    tnh = n // _POOL_CHAINS
    b1 = b1_ref[...]
    w2, b2 = w2_ref[...], b2_ref[...]
    w3 = w3_ref[...]

    def feat_chain(j, w1b, lo):
        xt = x_ref[j, :, pl.ds(lo, tnh)]
        h1 = jnp.maximum(
            jnp.dot(w1b, xt, preferred_element_type=_F32) + b1, 0.0)
        pf_ref[j, :, pl.ds(lo, tnh)] = h1
        h2 = jnp.maximum(
            jnp.dot(w2, h1, preferred_element_type=_F32) + b2, 0.0)
        return [jnp.max(
            jnp.dot(w3[c * _MC:(c + 1) * _MC, :], h2,
                    preferred_element_type=_F32),
            axis=1, keepdims=True) for c in range(nc)]

    cols = []
    for j in range(bs):
        w1b = w1b_ref[j]
        ms = [feat_chain(j, w1b, q * tnh) for q in range(_POOL_CHAINS)]
        cols.append(jnp.concatenate(
            [functools.reduce(jnp.maximum, parts)
             for parts in zip(*ms)], axis=0))
    g2 = jnp.concatenate(cols, axis=1) + b3_ref[...]

    gb = (jnp.dot(wg_ref[...], g2, preferred_element_type=_F32)
          + bh1_ref[...])

    wl = wl_ref[...]
    wh2, bh2 = wh2_ref[...], bh2_ref[...]
    wh3, bh3 = wh3_ref[...], bh3_ref[...]
    wh4, bh4 = wh4_ref[...], bh4_ref[...]

    def head_chain(j, gbj, lo):
        pf = pf_ref[j, :, pl.ds(lo, ht)]
        h = jnp.maximum(
            jnp.dot(wl, pf, preferred_element_type=_F32) + gbj, 0.0)
        h = jnp.maximum(
            jnp.dot(wh2, h, preferred_element_type=_F32) + bh2, 0.0)
        h = jnp.maximum(
            jnp.dot(wh3, h, preferred_element_type=_F32) + bh3, 0.0)
        o_ref[j, :, pl.ds(lo, ht)] = jnp.maximum(
            jnp.dot(wh4, h, preferred_element_type=_F32) + bh4, 0.0)

    for j in range(bs):
        gbj = gb[:, j:j + 1]
        for i in range(nht):
            head_chain(j, gbj, i * ht)


def kernel(x, stn_conv1_w, stn_conv1_b, stn_conv2_w, stn_conv2_b,
           stn_conv3_w, stn_conv3_b, stn_fc1_w, stn_fc1_b, stn_fc2_w,
           stn_fc2_b, stn_fc3_w, stn_fc3_b, feat_conv1_w, feat_conv1_b,
           feat_conv2_w, feat_conv2_b, feat_conv3_w, feat_conv3_b,
           head_conv1_wg, head_conv1_wl, head_conv1_b, head_conv2_w,
           head_conv2_b, head_conv3_w, head_conv3_b, head_conv4_w,
           head_conv4_b):
    B, C, N = x.shape
    k = head_conv4_w.shape[0]
    ht = _HEAD_TILE if N % _HEAD_TILE == 0 else N
    nht = N // ht
    bs = 1

    whl = head_conv1_wl

    fw1, fb1 = stn_fc1_w.T, stn_fc1_b.reshape(-1, 1)
    fw2, fb2 = stn_fc2_w.T, stn_fc2_b.reshape(-1, 1)
    fw3, fb3 = stn_fc3_w.T, stn_fc3_b.reshape(-1, 1)

    stn_in = (stn_conv1_w, stn_conv1_b, stn_conv2_w, stn_conv2_b,
              stn_conv3_w, stn_conv3_b, fw1, fb1, fw2, fb2, fw3, fb3)
    raw = pl.pallas_call(
        functools.partial(_stn_kernel, n=N, bs=bs),
        out_shape=jax.ShapeDtypeStruct((B, 36, 1), _F32),
        grid=(B // bs,),
        in_specs=[pl.BlockSpec((bs, C, N), lambda b: (b, 0, 0))]
        + [_const_spec(a) for a in stn_in],
        out_specs=pl.BlockSpec((bs, 36, 1), lambda b: (b, 0, 0)),
        compiler_params=pltpu.CompilerParams(
            dimension_semantics=("parallel",)),
    )(x, *stn_in)

    trans = raw[:, :, 0].reshape(B, C, C) + jnp.eye(C, dtype=_F32)[None]
    w1b = jnp.einsum("oc,bjc->boj", feat_conv1_w, trans)

    fh_in = (feat_conv1_b, feat_conv2_w, feat_conv2_b, feat_conv3_w,
             feat_conv3_b, head_conv1_wg, head_conv1_b, whl, head_conv2_w,
             head_conv2_b, head_conv3_w, head_conv3_b, head_conv4_w,
             head_conv4_b)
    out = pl.pallas_call(
        functools.partial(_feat_head_kernel, n=N, ht=ht, nht=nht, bs=bs),
        out_shape=jax.ShapeDtypeStruct((B, k, N), _F32),
        grid=(B // bs,),
        in_specs=[pl.BlockSpec((bs, C, N), lambda b: (b, 0, 0)),
                  pl.BlockSpec((bs, 64, C), lambda b: (b, 0, 0))]
        + [_const_spec(a) for a in fh_in],
        out_specs=pl.BlockSpec((bs, k, N), lambda b: (b, 0, 0)),
        scratch_shapes=[pltpu.VMEM((bs, 64, N), _F32)],
        compiler_params=pltpu.CompilerParams(
            dimension_semantics=("parallel",)),
    )(x, w1b, *fh_in)
    return out


# speedup vs baseline: 1.0132x; 1.0087x over previous
import functools

import jax
import jax.numpy as jnp
from jax import lax
from jax.experimental import pallas as pl
from jax.experimental.pallas import tpu as pltpu

_F32 = jnp.float32
_BF16 = jnp.bfloat16

_MC = 256
_POOL_CHAINS = 2
_HEAD_TILE = 16384


def _const_spec(a):
    return pl.BlockSpec(a.shape, lambda b: (0,) * a.ndim)


def _stn_kernel(x_ref, w1_ref, b1_ref, w2_ref, b2_ref, w3_ref, b3_ref,
                fw1_ref, fb1_ref, fw2_ref, fb2_ref, fw3_ref, fb3_ref,
                o_ref, *, n, bs):
    nc = 1024 // _MC
    tnh = n // _POOL_CHAINS
    w1, b1 = w1_ref[...], b1_ref[...]
    w2, b2 = w2_ref[...], b2_ref[...]
    w3 = w3_ref[...]

    def chain(j, lo):
        xt = x_ref[j, :, pl.ds(lo, tnh)]
        h1 = jnp.maximum(
            jnp.dot(w1, xt, preferred_element_type=_F32) + b1, 0.0)
        h2 = jnp.maximum(
            jnp.dot(w2, h1, preferred_element_type=_F32) + b2, 0.0)
        return [jnp.max(
            jnp.dot(w3[c * _MC:(c + 1) * _MC, :], h2,
                    preferred_element_type=_F32),
            axis=1, keepdims=True) for c in range(nc)]

    cols = []
    for j in range(bs):
        ms = [chain(j, q * tnh) for q in range(_POOL_CHAINS)]
        cols.append(jnp.concatenate(
            [functools.reduce(jnp.maximum, parts)
             for parts in zip(*ms)], axis=0))
    g = jnp.concatenate(cols, axis=1)
    g = jnp.maximum(g + b3_ref[...], 0.0)

    h = jnp.maximum(
        jnp.dot(fw1_ref[...], g, preferred_element_type=_F32)
        + fb1_ref[...], 0.0)
    h = jnp.maximum(
        jnp.dot(fw2_ref[...], h, preferred_element_type=_F32)
        + fb2_ref[...], 0.0)
    r = (jnp.dot(fw3_ref[...], h, preferred_element_type=_F32)
         + fb3_ref[...])
    for j in range(bs):
        o_ref[j] = r[:, j:j + 1]


def _feat_head_kernel(x_ref, w1b_ref, b1_ref, w2_ref, b2_ref, w3_ref, b3_ref,
                      wg_ref, bh1_ref, wl_ref, wh2_ref, bh2_ref,
                      wh3_ref, bh3_ref, wh4_ref, bh4_ref,
                      o_ref, pf_ref, *, n, ht, nht, bs):
    nc = 1024 // _MC
    tnh = n // _POOL_CHAINS
    b1 = b1_ref[...]
    w2, b2 = w2_ref[...], b2_ref[...]
    w3 = w3_ref[...]

    def feat_chain(j, w1b, lo):
        xt = x_ref[j, :, pl.ds(lo, tnh)]
        h1 = jnp.maximum(
            jnp.dot(w1b, xt, preferred_element_type=_F32) + b1, 0.0)
        pf_ref[j, :, pl.ds(lo, tnh)] = h1
        h2 = jnp.maximum(
            jnp.dot(w2, h1, preferred_element_type=_F32) + b2, 0.0)
        return [jnp.max(
            jnp.dot(w3[c * _MC:(c + 1) * _MC, :], h2,
                    preferred_element_type=_F32),
            axis=1, keepdims=True) for c in range(nc)]

    cols = []
    for j in range(bs):
        w1b = w1b_ref[j]
        ms = [feat_chain(j, w1b, q * tnh) for q in range(_POOL_CHAINS)]
        cols.append(jnp.concatenate(
            [functools.reduce(jnp.maximum, parts)
             for parts in zip(*ms)], axis=0))
    g2 = jnp.concatenate(cols, axis=1) + b3_ref[...]

    gb = (jnp.dot(wg_ref[...], g2, preferred_element_type=_F32)
          + bh1_ref[...])

    wl = wl_ref[...]
    wh2, bh2 = wh2_ref[...], bh2_ref[...]
    wh3, bh3 = wh3_ref[...], bh3_ref[...]
    wh4, bh4 = wh4_ref[...], bh4_ref[...]

    def head_chain(j, gbj, lo):
        pf = pf_ref[j, :, pl.ds(lo, ht)]
        h = jnp.maximum(
            jnp.dot(wl, pf, preferred_element_type=_F32) + gbj, 0.0)
        h = jnp.maximum(
            jnp.dot(wh2, h, preferred_element_type=_F32) + bh2, 0.0)
        h = jnp.maximum(
            jnp.dot(wh3, h, preferred_element_type=_F32) + bh3, 0.0)
        o_ref[j, :, pl.ds(lo, ht)] = jnp.maximum(
            jnp.dot(wh4, h, preferred_element_type=_F32) + bh4, 0.0)

    for j in range(bs):
        gbj = gb[:, j:j + 1]
        for i in range(nht):
            head_chain(j, gbj, i * ht)


def kernel(x, stn_conv1_w, stn_conv1_b, stn_conv2_w, stn_conv2_b,
           stn_conv3_w, stn_conv3_b, stn_fc1_w, stn_fc1_b, stn_fc2_w,
           stn_fc2_b, stn_fc3_w, stn_fc3_b, feat_conv1_w, feat_conv1_b,
           feat_conv2_w, feat_conv2_b, feat_conv3_w, feat_conv3_b,
           head_conv1_wg, head_conv1_wl, head_conv1_b, head_conv2_w,
           head_conv2_b, head_conv3_w, head_conv3_b, head_conv4_w,
           head_conv4_b):
    B, C, N = x.shape
    k = head_conv4_w.shape[0]
    ht = _HEAD_TILE if N % _HEAD_TILE == 0 else N
    nht = N // ht
    bs = 1

    whl = head_conv1_wl

    fw1, fb1 = stn_fc1_w.T, stn_fc1_b.reshape(-1, 1)
    fw2, fb2 = stn_fc2_w.T, stn_fc2_b.reshape(-1, 1)
    fw3, fb3 = stn_fc3_w.T, stn_fc3_b.reshape(-1, 1)

    stn_in = (stn_conv1_w, stn_conv1_b, stn_conv2_w, stn_conv2_b,
              stn_conv3_w, stn_conv3_b, fw1, fb1, fw2, fb2, fw3, fb3)
    raw = pl.pallas_call(
        functools.partial(_stn_kernel, n=N, bs=bs),
        out_shape=jax.ShapeDtypeStruct((B, 36, 1), _F32),
        grid=(B // bs,),
        in_specs=[pl.BlockSpec((bs, C, N), lambda b: (b, 0, 0))]
        + [_const_spec(a) for a in stn_in],
        out_specs=pl.BlockSpec((bs, 36, 1), lambda b: (b, 0, 0)),
        compiler_params=pltpu.CompilerParams(
            dimension_semantics=("parallel",)),
    )(x, *stn_in)

    trans = raw[:, :, 0].reshape(B, C, C) + jnp.eye(C, dtype=_F32)[None]
    w1b = jnp.einsum("oc,bjc->boj", feat_conv1_w, trans)

    fh_in = (feat_conv1_b, feat_conv2_w, feat_conv2_b, feat_conv3_w,
             feat_conv3_b, head_conv1_wg, head_conv1_b, whl, head_conv2_w,
             head_conv2_b, head_conv3_w, head_conv3_b, head_conv4_w,
             head_conv4_b)
    out = pl.pallas_call(
        functools.partial(_feat_head_kernel, n=N, ht=ht, nht=nht, bs=bs),
        out_shape=jax.ShapeDtypeStruct((B, k, N), _F32),
        grid=(B // bs,),
        in_specs=[pl.BlockSpec((bs, C, N), lambda b: (b, 0, 0)),
                  pl.BlockSpec((bs, 64, C), lambda b: (b, 0, 0))]
        + [_const_spec(a) for a in fh_in],
        out_specs=pl.BlockSpec((bs, k, N), lambda b: (b, 0, 0)),
        scratch_shapes=[pltpu.VMEM((bs, 64, N), _F32)],
        compiler_params=pltpu.CompilerParams(
            dimension_semantics=("parallel",)),
    )(x, w1b, *fh_in)
    return out
